# Initial kernel scaffold; baseline (speedup 1.0000x reference)
import jax
import jax.numpy as jnp
from jax.experimental import pallas as pl


def kernel(emb_s, edge_index, adj_values, position_ids, sids, pos, negs, emb_p_w, proj_W, proj_b, W0, b0, W1, b1, out_W, out_b):
    raise NotImplementedError("write your pallas kernel here")


# speedup vs baseline: 2.9573x; 2.9573x over previous
import functools

import jax
import jax.numpy as jnp
from jax import lax
from jax.experimental import pallas as pl
from jax.experimental.pallas import tpu as pltpu
from jax.experimental.pallas import tpu_sc as plsc

N = 10000
E = 320000
D = 128
PD = 64
DEPTH = 16
B = 1024
K = 32
TEMP = 0.5
LAMBDA_1 = 1e-05

NC = 2
NS = 16
L = 16
NW = NC * NS

CH = 128
EP = ((E + NW * CH - 1) // (NW * CH)) * (NW * CH)
EW = EP // NW
NCH = EW // CH
NP = 10240
RT = NP // NS

IDX = 2 * B + K * B
IDXP = ((IDX + NW * CH - 1) // (NW * CH)) * (NW * CH)
GW = IDXP // NW
GCH = GW // CH

BLK = 2000
GRID = N // BLK


def _rowmat(a, w):
    return lax.dot_general(a, w, (((1,), (1,)), ((), ())),
                           preferred_element_type=jnp.float32)


def _tc_prep(emb_s, pids2d, emb_p_w, proj_W, proj_b2, W0, b02):
    def body(es_ref, pid_ref, epw_ref, pw_ref, pb_ref, w0_ref, b0_ref,
             x_ref, h_ref):
        pid = pid_ref[...]
        io = lax.broadcasted_iota(jnp.int32, (BLK, DEPTH), 1)
        oh = (io == pid).astype(jnp.float32)
        ep = lax.dot_general(oh, epw_ref[...], (((1,), (0,)), ((), ())),
                             preferred_element_type=jnp.float32)
        cat = jnp.concatenate([es_ref[...], ep], axis=1)
        x = _rowmat(cat, pw_ref[...]) + pb_ref[...]
        x_ref[...] = x
        h_ref[...] = jnp.maximum(_rowmat(x, w0_ref[...]) + b0_ref[...], 0.0)

    return pl.pallas_call(
        body,
        grid=(GRID,),
        in_specs=[
            pl.BlockSpec((BLK, D), lambda i: (i, 0)),
            pl.BlockSpec((BLK, 1), lambda i: (i, 0)),
            pl.BlockSpec((DEPTH, PD), lambda i: (0, 0)),
            pl.BlockSpec((D, D + PD), lambda i: (0, 0)),
            pl.BlockSpec((1, D), lambda i: (0, 0)),
            pl.BlockSpec((D, D), lambda i: (0, 0)),
            pl.BlockSpec((1, D), lambda i: (0, 0)),
        ],
        out_specs=[
            pl.BlockSpec((BLK, D), lambda i: (i, 0)),
            pl.BlockSpec((BLK, D), lambda i: (i, 0)),
        ],
        out_shape=[
            jax.ShapeDtypeStruct((N, D), jnp.float32),
            jax.ShapeDtypeStruct((N, D), jnp.float32),
        ],
    )(emb_s, pids2d, emb_p_w, proj_W, proj_b2, W0, b02)


def _tc_mid(x, acc, W, b2):
    def body(x_ref, a_ref, w_ref, b_ref, x1_ref, h_ref):
        x1 = x_ref[...] + a_ref[0] + a_ref[1]
        x1_ref[...] = x1
        h_ref[...] = jnp.maximum(_rowmat(x1, w_ref[...]) + b_ref[...], 0.0)

    return pl.pallas_call(
        body,
        grid=(GRID,),
        in_specs=[
            pl.BlockSpec((BLK, D), lambda i: (i, 0)),
            pl.BlockSpec((NC, BLK, D), lambda i: (0, i, 0)),
            pl.BlockSpec((D, D), lambda i: (0, 0)),
            pl.BlockSpec((1, D), lambda i: (0, 0)),
        ],
        out_specs=[
            pl.BlockSpec((BLK, D), lambda i: (i, 0)),
            pl.BlockSpec((BLK, D), lambda i: (i, 0)),
        ],
        out_shape=[
            jax.ShapeDtypeStruct((N, D), jnp.float32),
            jax.ShapeDtypeStruct((N, D), jnp.float32),
        ],
    )(x, acc, W, b2)


def _tc_out(x1, acc, out_W, out_b2):
    def body(x_ref, a_ref, w_ref, b_ref, y_ref):
        x2 = x_ref[...] + a_ref[0] + a_ref[1]
        o = _rowmat(x2, w_ref[...]) + b_ref[...]
        nrm = jnp.sqrt(jnp.sum(o * o, axis=1, keepdims=True))
        y_ref[...] = o / jnp.maximum(nrm, 1e-8)

    return pl.pallas_call(
        body,
        grid=(GRID,),
        in_specs=[
            pl.BlockSpec((BLK, D), lambda i: (i, 0)),
            pl.BlockSpec((NC, BLK, D), lambda i: (0, i, 0)),
            pl.BlockSpec((D, D), lambda i: (0, 0)),
            pl.BlockSpec((1, D), lambda i: (0, 0)),
        ],
        out_specs=[pl.BlockSpec((BLK, D), lambda i: (i, 0))],
        out_shape=[jax.ShapeDtypeStruct((N, D), jnp.float32)],
    )(x1, acc, out_W, out_b2)[0]


def _sc_spmm(h, src, dst, vals, zeros):
    mesh = plsc.VectorSubcoreMesh(core_axis_name="c", subcore_axis_name="s")

    @functools.partial(
        pl.kernel,
        mesh=mesh,
        out_type=jax.ShapeDtypeStruct((NC, NP, D), jnp.float32),
        scratch_types=[
            pltpu.VMEM((CH,), jnp.int32),
            pltpu.VMEM((CH,), jnp.int32),
            pltpu.VMEM((CH,), jnp.float32),
            pltpu.VMEM((CH, D), jnp.float32),
            pltpu.VMEM_SHARED((NP, D), jnp.float32),
            pltpu.SemaphoreType.DMA,
        ],
    )
    def k(h_hbm, src_hbm, dst_hbm, val_hbm, z_hbm, out_hbm,
          srcb, dstb, valb, rows, acc, sem):
        c = lax.axis_index("c")
        s = lax.axis_index("s")
        pltpu.sync_copy(z_hbm.at[pl.ds(s * RT, RT)], acc.at[pl.ds(s * RT, RT)])
        plsc.subcore_barrier()

        base = (c * NS + s) * EW

        def chunk(g, carry):
            off = base + g * CH
            pltpu.sync_copy(src_hbm.at[pl.ds(off, CH)], srcb)
            pltpu.sync_copy(dst_hbm.at[pl.ds(off, CH)], dstb)
            pltpu.sync_copy(val_hbm.at[pl.ds(off, CH)], valb)
            pltpu.async_copy(h_hbm.at[srcb], rows, sem).wait()
            for gg in range(CH // L):
                v16 = valb[pl.ds(gg * L, L)]
                for e in range(L):
                    lane = jnp.full((L, 1), e, jnp.int32)
                    ve = lax.gather(
                        v16, lane,
                        lax.GatherDimensionNumbers(
                            offset_dims=(), collapsed_slice_dims=(0,),
                            start_index_map=(0,)),
                        (1,), mode=lax.GatherScatterMode.PROMISE_IN_BOUNDS)
                    r = gg * L + e
                    for q in range(D // L):
                        sl = (r, pl.ds(q * L, L))
                        rows[sl] = rows[sl] * ve
            pltpu.sync_copy(rows, acc.at[dstb], add=True)
            return carry

        lax.fori_loop(0, NCH, chunk, 0)
        plsc.subcore_barrier()
        pltpu.sync_copy(acc.at[pl.ds(s * RT, RT)],
                        out_hbm.at[c, pl.ds(s * RT, RT)])

    return k(h, src, dst, vals, zeros)


def _sc_gather(y, idx):
    mesh = plsc.VectorSubcoreMesh(core_axis_name="c", subcore_axis_name="s")

    @functools.partial(
        pl.kernel,
        mesh=mesh,
        out_type=jax.ShapeDtypeStruct((IDXP, D), jnp.float32),
        scratch_types=[
            pltpu.VMEM((CH,), jnp.int32),
            pltpu.VMEM((CH, D), jnp.float32),
            pltpu.SemaphoreType.DMA,
        ],
    )
    def k(y_hbm, idx_hbm, out_hbm, idxb, rows, sem):
        c = lax.axis_index("c")
        s = lax.axis_index("s")
        base = (c * NS + s) * GW
        for t in range(GCH):
            off = base + t * CH
            pltpu.sync_copy(idx_hbm.at[pl.ds(off, CH)], idxb)
            pltpu.async_copy(y_hbm.at[idxb], rows, sem).wait()
            pltpu.sync_copy(rows, out_hbm.at[pl.ds(off, CH)])

    return k(y, idx)


def _tc_loss(R, emb_p_w, proj_W, proj_b2, W0, b02, W1, b12, out_W, out_b2):
    def body(r_ref, epw, pw, pb, w0, b0, w1, b1, ow, ob,
             lo_ref, lcl_ref, lrg_ref):
        ys = r_ref[pl.ds(0, B), :]
        yp = r_ref[pl.ds(B, B), :]
        ps = jnp.sum(ys * yp, axis=1, keepdims=True)
        eps_ = jnp.exp(ps / TEMP)
        total = 0.0
        for kk in range(K):
            nk = r_ref[pl.ds(2 * B + kk * B, B), :]
            ns = jnp.sum(ys * nk, axis=1, keepdims=True)
            l = -jnp.log(eps_ / (eps_ + jnp.exp(ns / TEMP) + 1e-08))
            total = total + jnp.sum(l)
        loss_cl = total / (B * K)
        reg = (jnp.sum(epw[...] ** 2) + jnp.sum(pw[...] ** 2)
               + jnp.sum(pb[...] ** 2) + jnp.sum(w0[...] ** 2)
               + jnp.sum(b0[...] ** 2) + jnp.sum(w1[...] ** 2)
               + jnp.sum(b1[...] ** 2) + jnp.sum(ow[...] ** 2)
               + jnp.sum(ob[...] ** 2))
        loss_reg = reg * LAMBDA_1
        lcl_ref[...] = jnp.reshape(loss_cl, (1, 1))
        lrg_ref[...] = jnp.reshape(loss_reg, (1, 1))
        lo_ref[...] = jnp.reshape(loss_cl + loss_reg, (1, 1))

    return pl.pallas_call(
        body,
        out_shape=[
            jax.ShapeDtypeStruct((1, 1), jnp.float32),
            jax.ShapeDtypeStruct((1, 1), jnp.float32),
            jax.ShapeDtypeStruct((1, 1), jnp.float32),
        ],
    )(R, emb_p_w, proj_W, proj_b2, W0, b02, W1, b12, out_W, out_b2)


def kernel(emb_s, edge_index, adj_values, position_ids, sids, pos, negs,
           emb_p_w, proj_W, proj_b, W0, b0, W1, b1, out_W, out_b):
    i32 = jnp.int32
    dst = edge_index[0].astype(i32)
    src = edge_index[1].astype(i32)
    vals = adj_values.astype(jnp.float32)

    pad = EP - E
    src_p = jnp.concatenate([src, jnp.zeros((pad,), i32)])
    dst_p = jnp.concatenate([dst, jnp.zeros((pad,), i32)])
    val_p = jnp.concatenate([vals, jnp.zeros((pad,), jnp.float32)])

    pids2d = position_ids.astype(i32).reshape(N, 1)
    proj_b2 = proj_b.reshape(1, D)
    b02 = b0.reshape(1, D)
    b12 = b1.reshape(1, D)
    out_b2 = out_b.reshape(1, D)
    zeros = jnp.zeros((NP, D), jnp.float32)

    cat_idx = jnp.concatenate([
        sids.astype(i32), pos.astype(i32), negs.astype(i32).reshape(-1),
        jnp.zeros((IDXP - IDX,), i32),
    ])

    x, h0 = _tc_prep(emb_s, pids2d, emb_p_w, proj_W, proj_b2, W0, b02)
    acc1 = _sc_spmm(h0, src_p, dst_p, val_p, zeros)
    x1, h1 = _tc_mid(x, acc1, W1, b12)
    acc2 = _sc_spmm(h1, src_p, dst_p, val_p, zeros)
    y = _tc_out(x1, acc2, out_W, out_b2)
    R = _sc_gather(y, cat_idx)
    lo, lcl, lrg = _tc_loss(R, emb_p_w, proj_W, proj_b2, W0, b02, W1, b12,
                            out_W, out_b2)
    return (lo[0, 0], lcl[0, 0], lrg[0, 0])
